# baseline (speedup 1.0000x reference)
import functools

import jax
import jax.numpy as jnp
from jax import lax
from jax.experimental import pallas as pl
from jax.experimental.pallas import tpu as pltpu
from jax.experimental.pallas import tpu_sc as plsc

_BATCH = 4096
_HIST = 50
_EMBED = 64

_NC = 2
_NS = 16
_NW = _NC * _NS
_BB = _BATCH // _NW
_LANES = 16

_mesh = plsc.VectorSubcoreMesh(core_axis_name="c", subcore_axis_name="s")


@functools.partial(
    pl.kernel,
    mesh=_mesh,
    out_type=jax.ShapeDtypeStruct((_HIST, 8, _NW, 8, 128), jnp.float32),
    compiler_params=pltpu.CompilerParams(
        use_tc_tiling_on_sc=False, needs_layout_passes=False),
    scratch_types=[
        pltpu.VMEM((_BB, _HIST), jnp.int32),
        pltpu.VMEM((_HIST, _BB), jnp.int32),
        pltpu.VMEM((2, _BB, _EMBED), jnp.float32),
        pltpu.VMEM((2, 8, 8, 128), jnp.float32),
        pltpu.SemaphoreType.DMA,
        pltpu.SemaphoreType.DMA,
        pltpu.SemaphoreType.DMA,
        pltpu.SemaphoreType.DMA,
    ],
)
def _gather(idx_hbm, table_hbm, out_hbm, idx_v, idxt_v, rows_v, t_v,
            g0, g1, w0, w1):
    wid = lax.axis_index("s") * _NC + lax.axis_index("c")
    bbase = wid * _BB
    pltpu.sync_copy(idx_hbm.at[pl.ds(bbase, _BB)], idx_v)

    iota = lax.iota(jnp.int32, _LANES)
    bvecs = [iota + bb * _LANES for bb in range(_BB // _LANES)]
    zero16 = jnp.zeros((_LANES,), jnp.int32)

    def idx_t_body(h, c):
        hvec = zero16 + h
        for bb in range(_BB // _LANES):
            idxt_v[h, pl.ds(bb * _LANES, _LANES)] = plsc.load_gather(
                idx_v, [bvecs[bb], hvec])
        return c
    lax.fori_loop(0, _HIST, idx_t_body, 0)

    gsem = (g0, g1)
    wsem = (w0, w1)
    kvecs = (zero16, zero16 + 1)

    def start_gather(h, k):
        return pltpu.async_copy(
            table_hbm.at[idxt_v.at[h]], rows_v.at[k], gsem[k])

    def wait_gather(h, k):
        pltpu.make_async_copy(
            table_hbm.at[idxt_v.at[h]], rows_v.at[k], gsem[k]).wait()

    def start_write(h, k):
        return pltpu.async_copy(
            t_v.at[k], out_hbm.at[h, pl.ds(0, 8), wid], wsem[k])

    def wait_write(h, k):
        pltpu.make_async_copy(
            t_v.at[k], out_hbm.at[h, pl.ds(0, 8), wid], wsem[k]).wait()

    def transpose_rows(k):
        kvec = kvecs[k]
        def dbody(d, c):
            dvec = zero16 + d
            dB = d >> 3
            d8 = d & 7
            for bb in range(_BB // _LANES):
                t_v[k, dB, d8, pl.ds(bb * _LANES, _LANES)] = plsc.load_gather(
                    rows_v, [kvec, bvecs[bb], dvec])
            return c
        lax.fori_loop(0, _EMBED, dbody, 0)

    start_gather(0, 0)

    def step(i, c):
        h0 = 2 * i
        h1 = h0 + 1
        start_gather(h1, 1)
        wait_gather(h0, 0)

        @pl.when(i > 0)
        def _():
            wait_write(h0, 0)
        transpose_rows(0)
        start_write(h0, 0)

        @pl.when(i < _HIST // 2 - 1)
        def _():
            start_gather(h0 + 2, 0)
        wait_gather(h1, 1)

        @pl.when(i > 0)
        def _():
            wait_write(h1, 1)
        transpose_rows(1)
        start_write(h1, 1)
        return c

    lax.fori_loop(0, _HIST // 2, step, 0)
    wait_write(_HIST - 2, 0)
    wait_write(_HIST - 1, 1)


def kernel(inputs, table):
    p = _gather(inputs.astype(jnp.int32), table)
    return p.transpose(2, 4, 0, 1, 3).reshape(_BATCH, _HIST, _EMBED)
